# Initial kernel scaffold; baseline (speedup 1.0000x reference)
import jax
import jax.numpy as jnp
from jax.experimental import pallas as pl


def kernel(input, target):
    raise NotImplementedError("write your pallas kernel here")


# speedup vs baseline: 38.5077x; 38.5077x over previous
import jax
import jax.numpy as jnp
from jax.experimental import pallas as pl
from jax.experimental.pallas import tpu as pltpu


def _tc_body(n_cls, blk, nb, nj, n_total, x_ref, t_ref, out_ref,
             cnt_ref, fn_ref, ces_ref):
    b = pl.program_id(0)
    j = pl.program_id(1)

    @pl.when(jnp.logical_and(b == 0, j == 0))
    def _init():
        cnt_ref[...] = jnp.zeros_like(cnt_ref)
        fn_ref[...] = jnp.zeros_like(fn_ref)
        ces_ref[...] = jnp.zeros_like(ces_ref)

    x = x_ref[0]
    t = t_ref[0]

    m = jnp.max(x, axis=0, keepdims=True)
    e = jnp.exp(x - m)
    s = jnp.sum(e, axis=0, keepdims=True)
    lse = m + jnp.log(s)

    cls = jax.lax.broadcasted_iota(jnp.int32, (n_cls, blk), 0)
    oh = cls == t
    xt = jnp.sum(jnp.where(oh, x, 0.0), axis=0, keepdims=True)
    ce = lse - xt

    am = jnp.min(jnp.where(x == m, cls, n_cls), axis=0, keepdims=True)
    wrong = am != t

    cnt_arr = jnp.where(oh, 1.0, 0.0).astype(jnp.float32)
    fn_arr = jnp.where(jnp.logical_and(oh, wrong), 1.0, 0.0).astype(jnp.float32)
    ces_arr = jnp.where(oh, ce, 0.0).astype(jnp.float32)

    for k in range(blk // 128):
        sl = slice(k * 128, (k + 1) * 128)
        cnt_ref[...] += cnt_arr[:, sl]
        fn_ref[...] += fn_arr[:, sl]
        ces_ref[...] += ces_arr[:, sl]

    @pl.when(jnp.logical_and(b == nb - 1, j == nj - 1))
    def _fin():
        cnt = jnp.sum(cnt_ref[...], axis=1, keepdims=True)
        fn = jnp.sum(fn_ref[...], axis=1, keepdims=True)
        ces = jnp.sum(ces_ref[...], axis=1, keepdims=True)
        gt_c = jnp.where(cnt > 0, cnt, 1.0)
        fn_c = jnp.where(fn > 0, fn, 1.0)
        loss = jnp.sum((fn_c / gt_c) * ces) / jnp.float32(n_total)
        out_ref[...] = jnp.full(out_ref.shape, loss, jnp.float32)


def kernel(input, target):
    nb, n_cls, h, w = input.shape
    hw = h * w
    blk = 4096
    nj = hw // blk
    n_total = nb * hw

    inp3 = input.reshape(nb, n_cls, hw)
    t3 = target.reshape(nb * nj, 1, blk)

    import functools
    body = functools.partial(_tc_body, n_cls, blk, nb, nj, n_total)

    out = pl.pallas_call(
        body,
        grid=(nb, nj),
        in_specs=[
            pl.BlockSpec((1, n_cls, blk), lambda b, j: (b, 0, j)),
            pl.BlockSpec((1, 1, blk), lambda b, j, _nj=nj: (b * _nj + j, 0, 0)),
        ],
        out_specs=pl.BlockSpec((8, 128), lambda b, j: (0, 0)),
        out_shape=jax.ShapeDtypeStruct((8, 128), jnp.float32),
        scratch_shapes=[
            pltpu.VMEM((n_cls, 128), jnp.float32),
            pltpu.VMEM((n_cls, 128), jnp.float32),
            pltpu.VMEM((n_cls, 128), jnp.float32),
        ],
        compiler_params=pltpu.CompilerParams(
            dimension_semantics=("arbitrary", "arbitrary"),
        ),
    )(inp3, t3)
    return out[0, 0]


# speedup vs baseline: 117.2176x; 3.0440x over previous
import functools

import jax
import jax.numpy as jnp
from jax.experimental import pallas as pl
from jax.experimental.pallas import tpu as pltpu


def _tc_body(n_cls, nb, nj, n_total, x_ref, t_ref, out_ref,
             cnt_ref, fn_ref, ces_ref):
    b = pl.program_id(0)
    j = pl.program_id(1)

    @pl.when(jnp.logical_and(b == 0, j == 0))
    def _init():
        cnt_ref[...] = jnp.zeros_like(cnt_ref)
        fn_ref[...] = jnp.zeros_like(fn_ref)
        ces_ref[...] = jnp.zeros_like(ces_ref)

    x = x_ref[0]
    t = t_ref[...]

    m = jnp.max(x, axis=0, keepdims=True)
    xm = x - m
    e = jnp.exp(xm)
    s = jnp.sum(e, axis=0, keepdims=True)
    lse = m + jnp.log(s)

    cls = jax.lax.broadcasted_iota(jnp.int32, x.shape, 0)
    ohf = (cls == t).astype(jnp.float32)
    xt = jnp.sum(x * ohf, axis=0, keepdims=True)
    ce = lse - xt

    am = jnp.min(jnp.where(xm == 0.0, cls, n_cls), axis=0, keepdims=True)
    wrongf = (am != t).astype(jnp.float32)

    cnt_ref[...] += ohf
    fn_ref[...] += ohf * wrongf
    ces_ref[...] += ohf * ce

    @pl.when(jnp.logical_and(b == nb - 1, j == nj - 1))
    def _fin():
        cnt = jnp.sum(cnt_ref[...], axis=(1, 2), keepdims=True)
        fn = jnp.sum(fn_ref[...], axis=(1, 2), keepdims=True)
        ces = jnp.sum(ces_ref[...], axis=(1, 2), keepdims=True)
        gt_c = jnp.where(cnt > 0, cnt, 1.0)
        fn_c = jnp.where(fn > 0, fn, 1.0)
        loss = jnp.sum((fn_c / gt_c) * ces) / jnp.float32(n_total)
        out_ref[...] = jnp.full(out_ref.shape, loss, jnp.float32)


def kernel(input, target):
    nb, n_cls, h, w = input.shape
    rows = 16
    nj = h // rows
    n_total = nb * h * w

    body = functools.partial(_tc_body, n_cls, nb, nj, n_total)

    out = pl.pallas_call(
        body,
        grid=(nb, nj),
        in_specs=[
            pl.BlockSpec((1, n_cls, rows, w), lambda b, j: (b, 0, j, 0)),
            pl.BlockSpec((1, rows, w), lambda b, j: (b, j, 0)),
        ],
        out_specs=pl.BlockSpec((8, 128), lambda b, j: (0, 0)),
        out_shape=jax.ShapeDtypeStruct((8, 128), jnp.float32),
        scratch_shapes=[
            pltpu.VMEM((n_cls, rows, w), jnp.float32),
            pltpu.VMEM((n_cls, rows, w), jnp.float32),
            pltpu.VMEM((n_cls, rows, w), jnp.float32),
        ],
        compiler_params=pltpu.CompilerParams(
            dimension_semantics=("arbitrary", "arbitrary"),
        ),
    )(input, target)
    return out[0, 0]
